# baseline (device time: 144405 ns/iter reference)
import jax
import jax.numpy as jnp
from jax import lax
from jax.experimental import pallas as pl
from jax.experimental.pallas import tpu as pltpu


def kernel(x, dest):
    m, n = x.shape

    my_x = lax.axis_index("x")
    keep = dest == my_x
    k = jnp.sum(keep.astype(jnp.int32))
    s = m - k
    kpos = jnp.cumsum(keep.astype(jnp.int32)) - 1
    spos = jnp.cumsum((~keep).astype(jnp.int32)) - 1
    slot = jnp.where(keep, kpos, k + spos)
    src_of = jnp.zeros((m,), jnp.int32).at[slot].set(
        jnp.arange(m, dtype=jnp.int32)
    )
    counts = jnp.stack([k, s]).astype(jnp.int32)

    def body(x_ref, src_ref, cnt_ref, out_ref, send_sem, recv_sem,
             local_sem):
        mx = lax.axis_index("x")
        peer = (1 - mx, lax.axis_index("y"), lax.axis_index("z"))
        kk = cnt_ref[0]
        ss = cnt_ref[1]
        rr = ss
        keep_base = jnp.where(mx == 0, 0, rr)
        remote_base = jnp.where(mx == 0, 0, m - ss)

        barrier_sem = pltpu.get_barrier_semaphore()
        pl.semaphore_signal(
            barrier_sem, inc=1, device_id=peer,
            device_id_type=pl.DeviceIdType.MESH,
        )
        pl.semaphore_wait(barrier_sem, 1)

        def row(ref, idx):
            return ref.at[pl.ds(pl.multiple_of(idx * n, n), n)]

        def rem_body(t, c):
            pltpu.make_async_remote_copy(
                src_ref=row(x_ref, src_ref[t]),
                dst_ref=row(out_ref, remote_base + (t - kk)),
                send_sem=send_sem,
                recv_sem=recv_sem,
                device_id=peer,
                device_id_type=pl.DeviceIdType.MESH,
            ).start()
            return c

        lax.fori_loop(kk, m, rem_body, jnp.int32(0))

        def loc_body(t, c):
            pltpu.make_async_copy(
                row(x_ref, src_ref[t]), row(out_ref, keep_base + t),
                local_sem,
            ).start()
            return c

        lax.fori_loop(0, kk, loc_body, jnp.int32(0))

        send_wait = pltpu.make_async_remote_copy(
            src_ref=row(x_ref, 0), dst_ref=row(out_ref, 0),
            send_sem=send_sem, recv_sem=recv_sem,
            device_id=peer, device_id_type=pl.DeviceIdType.MESH,
        )
        local_wait = pltpu.make_async_copy(
            row(x_ref, 0), row(out_ref, 0), local_sem
        )

        def drain(count, wait):
            def f8(i, c):
                for _ in range(8):
                    wait()
                return c

            def f1(i, c):
                wait()
                return c

            lax.fori_loop(0, count // 8, f8, jnp.int32(0))
            lax.fori_loop(0, count % 8, f1, jnp.int32(0))

        drain(ss, send_wait.wait_send)
        drain(kk, local_wait.wait)
        drain(rr, send_wait.wait_recv)

    out_flat = pl.pallas_call(
        body,
        out_shape=jax.ShapeDtypeStruct((m * n,), x.dtype),
        in_specs=[
            pl.BlockSpec(memory_space=pltpu.VMEM),
            pl.BlockSpec(memory_space=pltpu.SMEM),
            pl.BlockSpec(memory_space=pltpu.SMEM),
        ],
        out_specs=pl.BlockSpec(memory_space=pltpu.VMEM),
        scratch_shapes=[
            pltpu.SemaphoreType.DMA,
            pltpu.SemaphoreType.DMA,
            pltpu.SemaphoreType.DMA,
        ],
        compiler_params=pltpu.CompilerParams(collective_id=0),
    )(x.reshape(m * n), src_of, counts)
    return out_flat.reshape(m, n)
